# baseline (speedup 1.0000x reference)
import functools

import jax
import jax.numpy as jnp
from jax import lax
from jax.experimental import pallas as pl
from jax.experimental.pallas import tpu as pltpu
from jax.experimental.pallas import tpu_sc as plsc

_NC = 2
_NS = 16
_NW = _NC * _NS
_L = 16
_NEG = 5
_CHUNK = 32
_NGRP = _CHUNK // _L


def _lane_shuffle(x, idx):
    return lax.gather(
        x, idx[:, None],
        lax.GatherDimensionNumbers(
            offset_dims=(), collapsed_slice_dims=(0,), start_index_map=(0,)),
        (1,),
        mode=lax.GatherScatterMode.PROMISE_IN_BOUNDS,
    )


def _fold16(vs, lane, perms):
    cur = list(vs)
    for s, p in zip((8, 4, 2, 1), perms):
        h = len(cur) // 2
        nxt = []
        for i in range(h):
            a = cur[i] + _lane_shuffle(cur[i], p)
            b = cur[i + h] + _lane_shuffle(cur[i + h], p)
            nxt.append(jnp.where((lane & s) == 0, a, b))
        cur = nxt
    return cur[0]


def _sc_body(nch, table_hbm, il_hbm, ol_hbm, ng_hbm, out_hbm,
             il_v, ol_v, ng_v, rows_v, acc_v, scores_v, sem0, sem1):
    wid = lax.axis_index("s") * _NC + lax.axis_index("c")
    bpw = nch * _CHUNK
    pltpu.sync_copy(il_hbm.at[pl.ds(wid * bpw, bpw)], il_v)
    pltpu.sync_copy(ol_hbm.at[pl.ds(wid * bpw, bpw)], ol_v)
    pltpu.sync_copy(ng_hbm.at[pl.ds(wid * bpw * _NEG, bpw * _NEG)], ng_v)
    lane = lax.iota(jnp.int32, _L)
    perms = [jnp.bitwise_xor(lane, s) for s in (8, 4, 2, 1)]

    def fire(c, par, sem):
        pltpu.async_copy(table_hbm.at[il_v.at[pl.ds(c * _CHUNK, _CHUNK)]],
                         rows_v.at[par, 0, pl.ds(0, _CHUNK)], sem)
        pltpu.async_copy(table_hbm.at[ol_v.at[pl.ds(c * _CHUNK, _CHUNK)]],
                         rows_v.at[par, 1, pl.ds(0, _CHUNK)], sem)
        for g in range(_NGRP):
            pltpu.async_copy(
                table_hbm.at[ng_v.at[pl.ds((c * _NGRP + g) * _L * _NEG,
                                           _L * _NEG)]],
                rows_v.at[par, 2 + g, pl.ds(0, _L * _NEG)], sem)

    def drain(c, par, sem):
        pltpu.make_async_copy(table_hbm.at[il_v.at[pl.ds(c * _CHUNK, _CHUNK)]],
                              rows_v.at[par, 0, pl.ds(0, _CHUNK)], sem).wait()
        pltpu.make_async_copy(table_hbm.at[ol_v.at[pl.ds(c * _CHUNK, _CHUNK)]],
                              rows_v.at[par, 1, pl.ds(0, _CHUNK)], sem).wait()
        for g in range(_NGRP):
            pltpu.make_async_copy(
                table_hbm.at[ng_v.at[pl.ds((c * _NGRP + g) * _L * _NEG,
                                           _L * _NEG)]],
                rows_v.at[par, 2 + g, pl.ds(0, _L * _NEG)], sem).wait()

    fire(0, 0, sem0)

    def compute(c, par):
        for grp in range(_NGRP):
            @plsc.parallel_loop(0, _L, unroll=2)
            def b_body(b16):
                b = grp * _L + b16
                inp = [rows_v[par, 0, b, pl.ds(_L * k, _L)] for k in range(8)]
                for j in range(6):
                    if j == 0:
                        src = lambda k: rows_v[par, 1, b, pl.ds(_L * k, _L)]
                    else:
                        src = lambda k: rows_v[par, 2 + grp, b16 * _NEG + (j - 1),
                                               pl.ds(_L * k, _L)]
                    acc = inp[0] * src(0)
                    for k in range(1, 8):
                        acc = acc + inp[k] * src(k)
                    acc_v[j, b, :] = acc

        @plsc.parallel_loop(0, _CHUNK // _L)
        def fold_grp(g):
            for j in range(6):
                vs = [acc_v[j, g * _L + i, :] for i in range(_L)]
                res = _fold16(vs, lane, perms)
                if j > 0:
                    res = 0.0 - res
                scores_v[j, pl.ds(c * _CHUNK + g * _L, _L)] = res

    def pair_body(c2, carry):
        c = c2 * 2
        fire(c + 1, 1, sem1)
        drain(c, 0, sem0)
        compute(c, 0)

        @pl.when(c + 2 < nch)
        def _():
            fire(c + 2, 0, sem0)

        drain(c + 1, 1, sem1)
        compute(c + 1, 1)
        return carry

    lax.fori_loop(0, nch // 2, pair_body, 0, unroll=False)
    pltpu.sync_copy(scores_v, out_hbm.at[wid])


def _sc_scores(table, il, ol, ng, batch, nch):
    mesh = plsc.VectorSubcoreMesh(
        core_axis_name="c", subcore_axis_name="s",
        num_cores=_NC, num_subcores=_NS,
    )
    bpw = nch * _CHUNK
    fn = pl.kernel(
        functools.partial(_sc_body, nch),
        out_type=jax.ShapeDtypeStruct((_NW, 6, bpw), jnp.float32),
        mesh=mesh,
        scratch_types=[
            pltpu.VMEM((bpw,), jnp.int32),
            pltpu.VMEM((bpw,), jnp.int32),
            pltpu.VMEM((bpw * _NEG,), jnp.int32),
            pltpu.VMEM((2, 2 + _NGRP, _L * _NEG, 128), jnp.float32),
            pltpu.VMEM((6, _CHUNK, _L), jnp.float32),
            pltpu.VMEM((6, bpw), jnp.float32),
            pltpu.SemaphoreType.DMA,
            pltpu.SemaphoreType.DMA,
        ],
    )
    return fn(table, il, ol, ng)


def _tc_reduce_body(batch, x_ref, o_ref):
    x = x_ref[...]
    ls = jnp.minimum(x, 0.0) - jnp.log(1.0 + jnp.exp(-jnp.abs(x)))
    o_ref[0, 0] = -jnp.sum(ls) / batch


def kernel(input_labels, out_labels, negatives, table):
    batch = input_labels.shape[0]
    assert batch % (_NW * _CHUNK) == 0
    nch = batch // (_NW * _CHUNK)

    il = input_labels.astype(jnp.int32)
    ol = out_labels.astype(jnp.int32)
    ng = negatives.astype(jnp.int32).reshape(batch * _NEG)

    scores = _sc_scores(table.astype(jnp.float32), il, ol, ng, batch, nch)

    loss = pl.pallas_call(
        functools.partial(_tc_reduce_body, batch),
        out_shape=jax.ShapeDtypeStruct((1, 1), jnp.float32),
        out_specs=pl.BlockSpec(memory_space=pltpu.SMEM),
    )(scores.reshape(batch * 6 // 128, 128))
    return loss[0, 0]


# speedup vs baseline: 1.2699x; 1.2699x over previous
import functools

import jax
import jax.numpy as jnp
from jax import lax
from jax.experimental import pallas as pl
from jax.experimental.pallas import tpu as pltpu
from jax.experimental.pallas import tpu_sc as plsc

_NC = 2
_NS = 16
_NW = _NC * _NS
_L = 16
_NEG = 5
_NIDX = _NEG + 2
_CHUNK = 32


def _lane_shuffle(x, idx):
    return lax.gather(
        x, idx[:, None],
        lax.GatherDimensionNumbers(
            offset_dims=(), collapsed_slice_dims=(0,), start_index_map=(0,)),
        (1,),
        mode=lax.GatherScatterMode.PROMISE_IN_BOUNDS,
    )


def _fold16(vs, lane, perms):
    cur = list(vs)
    for s, p in zip((8, 4, 2, 1), perms):
        h = len(cur) // 2
        nxt = []
        for i in range(h):
            a = cur[i] + _lane_shuffle(cur[i], p)
            b = cur[i + h] + _lane_shuffle(cur[i + h], p)
            nxt.append(jnp.where((lane & s) == 0, a, b))
        cur = nxt
    return cur[0]


def _sc_body(nch, table_hbm, il_hbm, ol_hbm, ng_hbm, out_hbm,
             idx_v, rows_v, acc_v, scores_v, sem0, sem1):
    wid = lax.axis_index("s") * _NC + lax.axis_index("c")
    bpw = nch * _CHUNK
    base = wid * bpw
    pltpu.sync_copy(il_hbm.at[pl.ds(base, bpw)], idx_v.at[0, 0])
    pltpu.sync_copy(ol_hbm.at[pl.ds(base, bpw)], idx_v.at[1, 0])
    nb = nch * _CHUNK * _NW
    for q in range(_NEG):
        pltpu.sync_copy(ng_hbm.at[pl.ds(q * nb + base, bpw)], idx_v.at[2 + q, 0])
    lane = lax.iota(jnp.int32, _L)
    perms = [jnp.bitwise_xor(lane, s) for s in (8, 4, 2, 1)]

    def fire(c, par, sem):
        for r in range(_NIDX):
            pltpu.async_copy(
                table_hbm.at[idx_v.at[r, 0, pl.ds(c * _CHUNK, _CHUNK)]],
                rows_v.at[par, r], sem)

    def drain(c, par, sem):
        for r in range(_NIDX):
            pltpu.make_async_copy(
                table_hbm.at[idx_v.at[r, 0, pl.ds(c * _CHUNK, _CHUNK)]],
                rows_v.at[par, r], sem).wait()

    fire(0, 0, sem0)

    def compute(c, par):
        @plsc.parallel_loop(0, _CHUNK, unroll=2)
        def b_body(b):
            inp = [rows_v[par, 0, b, pl.ds(_L * k, _L)] for k in range(8)]
            for j in range(6):
                acc = inp[0] * rows_v[par, j + 1, b, pl.ds(0, _L)]
                for k in range(1, 8):
                    acc = acc + inp[k] * rows_v[par, j + 1, b, pl.ds(_L * k, _L)]
                acc_v[j, b, :] = acc

        @plsc.parallel_loop(0, _CHUNK // _L)
        def fold_grp(g):
            for j in range(6):
                vs = [acc_v[j, g * _L + i, :] for i in range(_L)]
                res = _fold16(vs, lane, perms)
                if j > 0:
                    res = 0.0 - res
                scores_v[j, pl.ds(c * _CHUNK + g * _L, _L)] = res

    def pair_body(c2, carry):
        c = c2 * 2
        fire(c + 1, 1, sem1)
        drain(c, 0, sem0)
        compute(c, 0)

        @pl.when(c + 2 < nch)
        def _():
            fire(c + 2, 0, sem0)

        drain(c + 1, 1, sem1)
        compute(c + 1, 1)
        return carry

    lax.fori_loop(0, nch // 2, pair_body, 0, unroll=False)
    pltpu.sync_copy(scores_v, out_hbm.at[wid])


def _sc_scores(table, il, ol, ng, batch, nch):
    mesh = plsc.VectorSubcoreMesh(
        core_axis_name="c", subcore_axis_name="s",
        num_cores=_NC, num_subcores=_NS,
    )
    bpw = nch * _CHUNK
    fn = pl.kernel(
        functools.partial(_sc_body, nch),
        out_type=jax.ShapeDtypeStruct((_NW, 6, bpw), jnp.float32),
        mesh=mesh,
        scratch_types=[
            pltpu.VMEM((_NIDX, 1, bpw), jnp.int32),
            pltpu.VMEM((2, _NIDX, _CHUNK, 128), jnp.float32),
            pltpu.VMEM((6, _CHUNK, _L), jnp.float32),
            pltpu.VMEM((6, bpw), jnp.float32),
            pltpu.SemaphoreType.DMA,
            pltpu.SemaphoreType.DMA,
        ],
    )
    return fn(table, il, ol, ng)


def _tc_reduce_body(batch, x_ref, o_ref):
    x = x_ref[...]
    ls = jnp.minimum(x, 0.0) - jnp.log(1.0 + jnp.exp(-jnp.abs(x)))
    o_ref[0, 0] = -jnp.sum(ls) / batch


def kernel(input_labels, out_labels, negatives, table):
    batch = input_labels.shape[0]
    assert batch % (_NW * _CHUNK) == 0
    nch = batch // (_NW * _CHUNK)

    il = input_labels.astype(jnp.int32)
    ol = out_labels.astype(jnp.int32)
    ng = negatives.astype(jnp.int32).T.reshape(batch * _NEG)

    scores = _sc_scores(table.astype(jnp.float32), il, ol, ng, batch, nch)

    loss = pl.pallas_call(
        functools.partial(_tc_reduce_body, batch),
        out_shape=jax.ShapeDtypeStruct((1, 1), jnp.float32),
        out_specs=pl.BlockSpec(memory_space=pltpu.SMEM),
    )(scores.reshape(batch * 6 // 128, 128))
    return loss[0, 0]
